# Initial kernel scaffold; baseline (speedup 1.0000x reference)
import jax
import jax.numpy as jnp
from jax.experimental import pallas as pl


def kernel(x, table):
    raise NotImplementedError("write your pallas kernel here")


# speedup vs baseline: 1.8650x; 1.8650x over previous
import functools

import jax
import jax.numpy as jnp
from jax import lax
from jax.experimental import pallas as pl
from jax.experimental.pallas import tpu as pltpu
from jax.experimental.pallas import tpu_sc as plsc


def _gather_kernel(n_rows, d, n_workers, chunk):
    n_chunks_per_w = n_rows // (n_workers * chunk)
    b_per_w = n_rows // n_workers
    mesh = plsc.VectorSubcoreMesh(core_axis_name="c", subcore_axis_name="s")

    @functools.partial(
        pl.kernel,
        mesh=mesh,
        out_type=jax.ShapeDtypeStruct((n_rows, d), jnp.float32),
        scratch_types=[
            pltpu.VMEM((chunk,), jnp.int32),
            pltpu.VMEM((chunk, d), jnp.float32),
            pltpu.SemaphoreType.DMA,
        ],
        compiler_params=pltpu.CompilerParams(use_tc_tiling_on_sc=False),
    )
    def k(idx_hbm, table_hbm, out_hbm, idx_v, rows_v, sem):
        cid = lax.axis_index("c")
        sid = lax.axis_index("s")
        wid = sid * 2 + cid
        base = wid * b_per_w

        def body(i, _):
            off = base + i * chunk
            pltpu.sync_copy(idx_hbm.at[pl.ds(off, chunk)], idx_v)
            pltpu.async_copy(table_hbm.at[idx_v], rows_v, sem).wait()
            pltpu.sync_copy(rows_v, out_hbm.at[pl.ds(off, chunk)])
            return 0

        lax.fori_loop(0, n_chunks_per_w, body, 0)

    return k


def kernel(x, table):
    b, f = x.shape
    v, d = table.shape
    n_rows = b * f
    idx = x.reshape(n_rows)
    out = _gather_kernel(n_rows, d, n_workers=32, chunk=1024)(idx, table)
    return out.reshape(b, f * d)


# speedup vs baseline: 1.9090x; 1.0236x over previous
import functools

import jax
import jax.numpy as jnp
from jax import lax
from jax.experimental import pallas as pl
from jax.experimental.pallas import tpu as pltpu
from jax.experimental.pallas import tpu_sc as plsc

_NW = 32
_NBUF = 3


def _gather_kernel(n_rows, d, chunk):
    b_per_w = n_rows // _NW
    n_chunks = b_per_w // chunk
    mesh = plsc.VectorSubcoreMesh(core_axis_name="c", subcore_axis_name="s")

    @functools.partial(
        pl.kernel,
        mesh=mesh,
        out_type=jax.ShapeDtypeStruct((n_rows, d), jnp.float32),
        scratch_types=[
            pltpu.VMEM((b_per_w,), jnp.int32),
            [pltpu.VMEM((chunk, d), jnp.float32) for _ in range(_NBUF)],
            [pltpu.SemaphoreType.DMA for _ in range(_NBUF)],
            [pltpu.SemaphoreType.DMA for _ in range(_NBUF)],
        ],
        compiler_params=pltpu.CompilerParams(use_tc_tiling_on_sc=False),
    )
    def k(idx_hbm, table_hbm, out_hbm, idx_all, rows, sem_g, sem_o):
        cid = lax.axis_index("c")
        sid = lax.axis_index("s")
        wid = sid * 2 + cid
        base = wid * b_per_w

        pltpu.sync_copy(idx_hbm.at[pl.ds(base, b_per_w)], idx_all)

        gathers = {}
        stores = {}

        def start_store(j):
            r = j % _NBUF
            gathers[j].wait()
            stores[j] = pltpu.async_copy(
                rows[r], out_hbm.at[pl.ds(base + j * chunk, chunk)], sem_o[r]
            )

        for i in range(n_chunks):
            r = i % _NBUF
            if i >= _NBUF:
                stores[i - _NBUF].wait()
            gathers[i] = pltpu.async_copy(
                table_hbm.at[idx_all.at[pl.ds(i * chunk, chunk)]], rows[r], sem_g[r]
            )
            if i >= 1:
                start_store(i - 1)
        start_store(n_chunks - 1)
        for j in range(max(0, n_chunks - _NBUF + 1), n_chunks):
            stores[j].wait()

    return k


def kernel(x, table):
    b, f = x.shape
    v, d = table.shape
    n_rows = b * f
    chunk = 1024
    idx = x.reshape(n_rows)
    out = _gather_kernel(n_rows, d, chunk)(idx, table)
    return out.reshape(b, f * d)
